# Initial kernel scaffold; baseline (speedup 1.0000x reference)
import jax
import jax.numpy as jnp
from jax.experimental import pallas as pl


def kernel(x, edge, W_gnn, b_gnn, W1, b1, W2, b2, Wk, bk):
    raise NotImplementedError("write your pallas kernel here")


# speedup vs baseline: 1.6034x; 1.6034x over previous
import jax
import jax.numpy as jnp
from jax import lax
from jax.experimental import pallas as pl
from jax.experimental.pallas import tpu as pltpu
from jax.experimental.pallas import tpu_sc as plsc

N = 10000
E = 160000
D_IN = 256
D_HID = 512
D_LAT = 256

NC = 2
NS = 16
NP = 2
DH = D_IN // NC

CK = 128
CH = 80
EPAD = NS * CH * CK

RANGE = 5120
ROWS_P = RANGE + CK
ZPT = ROWS_P // NS
OPT = RANGE // NS

CHD = CH // NC
ROWS_D = 10368
ZPT_D = ROWS_D // NS
OPT_D = 10240 // NS


def _chunks(total, step):
    out = []
    off = 0
    while off < total:
        sz = min(step, total - off)
        out.append((off, sz))
        off += sz
    return tuple(out)


ZCH = _chunks(ZPT, CK)
OCH = _chunks(OPT, CK)
ZCHD = _chunks(ZPT_D, CK)
OCHD = _chunks(OPT_D, CK)


def _agg_body(x2_hbm, src_hbm, dst_hbm, zrow_hbm, agg_hbm,
              src_v, dst_v, rows_v, zrow_v, agg_s):
    c = lax.axis_index("c")
    s = lax.axis_index("s")

    pltpu.sync_copy(src_hbm.at[c * NS + s], src_v)
    pltpu.sync_copy(zrow_hbm, zrow_v)

    for p in range(NP):
        base = p * RANGE
        pltpu.sync_copy(dst_hbm.at[p * NS + s], dst_v)

        for off, sz in ZCH:
            r0 = s * ZPT + off
            pltpu.sync_copy(zrow_v.at[pl.ds(0, sz)], agg_s.at[pl.ds(r0, sz)])
        plsc.subcore_barrier()

        def body(ch, carry):
            pltpu.sync_copy(x2_hbm.at[src_v.at[ch]], rows_v)
            pltpu.sync_copy(rows_v, agg_s.at[dst_v.at[ch]], add=True)
            return carry

        lax.fori_loop(0, CH, body, None)
        plsc.subcore_barrier()

        for off, sz in OCH:
            lr = s * OPT + off
            pltpu.sync_copy(agg_s.at[pl.ds(lr, sz)],
                            agg_hbm.at[c, pl.ds(base + lr, sz)])
        if p + 1 < NP:
            plsc.subcore_barrier()


def _deg_body(dst_hbm, ones_hbm, zrow_hbm, deg_hbm,
              dst_v, ones_v, zrow_v, deg_s):
    c = lax.axis_index("c")
    s = lax.axis_index("s")

    pltpu.sync_copy(dst_hbm.at[s * NC + c], dst_v)
    pltpu.sync_copy(ones_hbm, ones_v)
    pltpu.sync_copy(zrow_hbm, zrow_v)

    for off, sz in ZCHD:
        r0 = s * ZPT_D + off
        pltpu.sync_copy(zrow_v.at[pl.ds(0, sz)], deg_s.at[pl.ds(r0, sz)])
    plsc.subcore_barrier()

    def body(ch, carry):
        pltpu.sync_copy(ones_v, deg_s.at[dst_v.at[ch]], add=True)
        return carry

    lax.fori_loop(0, CHD, body, None)
    plsc.subcore_barrier()

    for off, sz in OCHD:
        lr = s * OPT_D + off
        pltpu.sync_copy(deg_s.at[pl.ds(lr, sz)],
                        deg_hbm.at[c, pl.ds(lr, sz)])


def _sc_aggregate(x2, src_i, dst_i, zrow_blk):
    mesh = plsc.VectorSubcoreMesh(core_axis_name="c", subcore_axis_name="s")
    f32 = jnp.float32
    run = pl.kernel(
        _agg_body,
        out_type=jax.ShapeDtypeStruct((NC, NP * RANGE, DH), f32),
        mesh=mesh,
        scratch_types=[
            pltpu.VMEM((CH, CK), jnp.int32),
            pltpu.VMEM((CH, CK), jnp.int32),
            pltpu.VMEM((CK, DH), f32),
            pltpu.VMEM((CK, DH), f32),
            pltpu.VMEM_SHARED((ROWS_P, DH), f32),
        ],
    )
    return run(x2, src_i, dst_i, zrow_blk)


def _sc_degree(dst_i, ones_blk, zrow_blk):
    mesh = plsc.VectorSubcoreMesh(core_axis_name="c", subcore_axis_name="s")
    f32 = jnp.float32
    run = pl.kernel(
        _deg_body,
        out_type=jax.ShapeDtypeStruct((NC, NS * OPT_D, DH), f32),
        mesh=mesh,
        scratch_types=[
            pltpu.VMEM((CHD, CK), jnp.int32),
            pltpu.VMEM((CK, DH), f32),
            pltpu.VMEM((CK, DH), f32),
            pltpu.VMEM_SHARED((ROWS_D, DH), f32),
        ],
    )
    return run(dst_i, ones_blk, zrow_blk)


def _leaky(t):
    return jnp.where(t >= 0, t, 0.01 * t)


_PREC = lax.Precision.HIGHEST


def _mlp_body(a0_ref, a1_ref, x_ref, d0_ref, d1_ref,
              wg_ref, bg_ref, w1_ref, b1_ref, w2_ref, b2_ref, z_ref):
    x = x_ref[...]
    agg = jnp.concatenate([a0_ref[...], a1_ref[...]], axis=1)
    deg = d0_ref[...] + d1_ref[...]
    h = (agg + x) / (deg + 1.0)
    t = jnp.dot(h, wg_ref[...], precision=_PREC,
                preferred_element_type=jnp.float32) + bg_ref[...]
    t = _leaky(t)
    t = jnp.dot(t, w1_ref[...], precision=_PREC,
                preferred_element_type=jnp.float32) + b1_ref[...]
    t = _leaky(t)
    z_ref[...] = jnp.dot(t, w2_ref[...], precision=_PREC,
                         preferred_element_type=jnp.float32) + b2_ref[...]


def _keys_body(x_ref, wk_ref, bk_ref, out_ref):
    out_ref[...] = jnp.dot(x_ref[...], wk_ref[...], precision=_PREC,
                           preferred_element_type=jnp.float32) + bk_ref[...]


_RB = 1000


def _mlp_call(a0, a1, x, d0, d1, Wg, bg, W1, b1, W2, b2):
    grid = (N // _RB,)
    row = lambda i: (i, 0)
    rep = lambda i: (0, 0)
    return pl.pallas_call(
        _mlp_body,
        grid=grid,
        in_specs=[
            pl.BlockSpec((_RB, DH), row),
            pl.BlockSpec((_RB, DH), row),
            pl.BlockSpec((_RB, D_IN), row),
            pl.BlockSpec((_RB, 1), row),
            pl.BlockSpec((_RB, 1), row),
            pl.BlockSpec((D_IN, D_HID), rep),
            pl.BlockSpec((1, D_HID), rep),
            pl.BlockSpec((D_HID, D_HID), rep),
            pl.BlockSpec((1, D_HID), rep),
            pl.BlockSpec((D_HID, D_LAT), rep),
            pl.BlockSpec((1, D_LAT), rep),
        ],
        out_specs=pl.BlockSpec((_RB, D_LAT), row),
        out_shape=jax.ShapeDtypeStruct((N, D_LAT), jnp.float32),
    )(a0, a1, x, d0, d1, Wg, bg, W1, b1, W2, b2)


def _keys_call(x, Wk, bk):
    grid = (N // _RB,)
    return pl.pallas_call(
        _keys_body,
        grid=grid,
        in_specs=[
            pl.BlockSpec((_RB, D_IN), lambda i: (i, 0)),
            pl.BlockSpec((D_IN, D_LAT), lambda i: (0, 0)),
            pl.BlockSpec((1, D_LAT), lambda i: (0, 0)),
        ],
        out_specs=pl.BlockSpec((_RB, D_LAT), lambda i: (i, 0)),
        out_shape=jax.ShapeDtypeStruct((N, D_LAT), jnp.float32),
    )(x, Wk, bk)


def kernel(x, edge, W_gnn, b_gnn, W1, b1, W2, b2, Wk, bk):
    src = edge[0].astype(jnp.int32)
    dst = edge[1].astype(jnp.int32)
    pad = EPAD - E
    srcp = jnp.concatenate([src, jnp.zeros((pad,), jnp.int32)])
    dstp = jnp.concatenate([dst, jnp.full((pad,), -1, jnp.int32)])
    x2 = x.reshape(N, NC, DH).reshape(N * NC, DH)
    src2 = (srcp[None, :] * NC
            + jnp.arange(NC, dtype=jnp.int32)[:, None]).reshape(NC * NS, CH, CK)
    lo = dstp[None, :] - (RANGE * jnp.arange(NP, dtype=jnp.int32))[:, None]
    dst3 = jnp.where((lo >= 0) & (lo < RANGE), lo, RANGE)
    dst3 = dst3.reshape(NP * NS, CH, CK)
    dstd = jnp.where(dstp < 0, N, dstp).reshape(NS * NC, CHD, CK)

    ones_blk = jnp.ones((CK, DH), jnp.float32)
    zrow_blk = jnp.zeros((CK, DH), jnp.float32)
    agg = _sc_aggregate(x2, src2, dst3, zrow_blk)
    deg = _sc_degree(dstd, ones_blk, zrow_blk)

    keys = _keys_call(x, Wk, bk.reshape(1, D_LAT))
    z = _mlp_call(agg[0, :N], agg[1, :N], x,
                  deg[0, :N, 0:1], deg[1, :N, 0:1],
                  W_gnn, b_gnn.reshape(1, D_HID),
                  W1, b1.reshape(1, D_HID),
                  W2, b2.reshape(1, D_LAT))
    return (z, keys)


# speedup vs baseline: 2.5892x; 1.6148x over previous
import jax
import jax.numpy as jnp
from jax import lax
from jax.experimental import pallas as pl
from jax.experimental.pallas import tpu as pltpu
from jax.experimental.pallas import tpu_sc as plsc

N = 10000
E = 160000
D_IN = 256
D_HID = 512
D_LAT = 256

NC = 2
NS = 16
DH = D_IN // NC

CK = 128
CH = 80
EPAD = NS * CH * CK

ROWS_A = 10368
ZPT = ROWS_A // NS
NOUT = 10240
OPT = NOUT // NS

CHD = CH // NC
ROWS_D = 10368
ZPT_D = ROWS_D // NS
OPT_D = NOUT // NS


def _chunks(total, step):
    out = []
    off = 0
    while off < total:
        sz = min(step, total - off)
        out.append((off, sz))
        off += sz
    return tuple(out)


OCH = _chunks(OPT, CK)
OCHD = _chunks(OPT_D, CK)


def _agg_body(x2_hbm, src_hbm, dst_hbm, zrow_hbm, agg_hbm,
              src_v, dst_v, rows_v, agg_s):
    c = lax.axis_index("c")
    s = lax.axis_index("s")

    pltpu.sync_copy(src_hbm.at[c * NS + s], src_v)
    pltpu.sync_copy(dst_hbm.at[s], dst_v)
    pltpu.sync_copy(zrow_hbm, agg_s.at[pl.ds(s * ZPT, ZPT)])
    plsc.subcore_barrier()

    def body(ch, carry):
        pltpu.sync_copy(x2_hbm.at[src_v.at[ch]], rows_v)
        pltpu.sync_copy(rows_v, agg_s.at[dst_v.at[ch]], add=True)
        return carry

    lax.fori_loop(0, CH, body, None)
    plsc.subcore_barrier()

    for off, sz in OCH:
        lr = s * OPT + off
        pltpu.sync_copy(agg_s.at[pl.ds(lr, sz)],
                        agg_hbm.at[c, pl.ds(lr, sz)])


def _deg_body(dst_hbm, ones_hbm, zrow_hbm, deg_hbm,
              dst_v, ones_v, deg_s):
    c = lax.axis_index("c")
    s = lax.axis_index("s")

    pltpu.sync_copy(dst_hbm.at[s * NC + c], dst_v)
    pltpu.sync_copy(ones_hbm, ones_v)
    pltpu.sync_copy(zrow_hbm, deg_s.at[pl.ds(s * ZPT_D, ZPT_D)])
    plsc.subcore_barrier()

    def body(ch, carry):
        pltpu.sync_copy(ones_v, deg_s.at[dst_v.at[ch]], add=True)
        return carry

    lax.fori_loop(0, CHD, body, None)
    plsc.subcore_barrier()

    for off, sz in OCHD:
        lr = s * OPT_D + off
        pltpu.sync_copy(deg_s.at[pl.ds(lr, sz)],
                        deg_hbm.at[c, pl.ds(lr, sz)])


def _sc_aggregate(x2, src_i, dst_i, zrow_blk):
    mesh = plsc.VectorSubcoreMesh(core_axis_name="c", subcore_axis_name="s")
    f32 = jnp.float32
    run = pl.kernel(
        _agg_body,
        out_type=jax.ShapeDtypeStruct((NC, NOUT, DH), f32),
        mesh=mesh,
        scratch_types=[
            pltpu.VMEM((CH, CK), jnp.int32),
            pltpu.VMEM((CH, CK), jnp.int32),
            pltpu.VMEM((CK, DH), f32),
            pltpu.VMEM_SHARED((ROWS_A, DH), f32),
        ],
    )
    return run(x2, src_i, dst_i, zrow_blk)


def _sc_degree(dst_i, ones_blk, zrow_blk):
    mesh = plsc.VectorSubcoreMesh(core_axis_name="c", subcore_axis_name="s")
    f32 = jnp.float32
    run = pl.kernel(
        _deg_body,
        out_type=jax.ShapeDtypeStruct((NC, NS * OPT_D, DH), f32),
        mesh=mesh,
        scratch_types=[
            pltpu.VMEM((CHD, CK), jnp.int32),
            pltpu.VMEM((CK, DH), f32),
            pltpu.VMEM_SHARED((ROWS_D, DH), f32),
        ],
    )
    return run(dst_i, ones_blk, zrow_blk)


def _leaky(t):
    return jnp.where(t >= 0, t, 0.01 * t)


_PREC = lax.Precision.HIGHEST


def _mlp_body(a0_ref, a1_ref, x_ref, d0_ref, d1_ref,
              wg_ref, bg_ref, w1_ref, b1_ref, w2_ref, b2_ref, z_ref):
    x = x_ref[...]
    agg = jnp.concatenate([a0_ref[...], a1_ref[...]], axis=1)
    deg = d0_ref[...] + d1_ref[...]
    h = (agg + x) / (deg + 1.0)
    t = jnp.dot(h, wg_ref[...], precision=_PREC,
                preferred_element_type=jnp.float32) + bg_ref[...]
    t = _leaky(t)
    t = jnp.dot(t, w1_ref[...], precision=_PREC,
                preferred_element_type=jnp.float32) + b1_ref[...]
    t = _leaky(t)
    z_ref[...] = jnp.dot(t, w2_ref[...], precision=_PREC,
                         preferred_element_type=jnp.float32) + b2_ref[...]


def _keys_body(x_ref, wk_ref, bk_ref, out_ref):
    out_ref[...] = jnp.dot(x_ref[...], wk_ref[...], precision=_PREC,
                           preferred_element_type=jnp.float32) + bk_ref[...]


_RB = 1000


def _mlp_call(a0, a1, x, d0, d1, Wg, bg, W1, b1, W2, b2):
    grid = (N // _RB,)
    row = lambda i: (i, 0)
    rep = lambda i: (0, 0)
    return pl.pallas_call(
        _mlp_body,
        grid=grid,
        in_specs=[
            pl.BlockSpec((_RB, DH), row),
            pl.BlockSpec((_RB, DH), row),
            pl.BlockSpec((_RB, D_IN), row),
            pl.BlockSpec((_RB, 1), row),
            pl.BlockSpec((_RB, 1), row),
            pl.BlockSpec((D_IN, D_HID), rep),
            pl.BlockSpec((1, D_HID), rep),
            pl.BlockSpec((D_HID, D_HID), rep),
            pl.BlockSpec((1, D_HID), rep),
            pl.BlockSpec((D_HID, D_LAT), rep),
            pl.BlockSpec((1, D_LAT), rep),
        ],
        out_specs=pl.BlockSpec((_RB, D_LAT), row),
        out_shape=jax.ShapeDtypeStruct((N, D_LAT), jnp.float32),
    )(a0, a1, x, d0, d1, Wg, bg, W1, b1, W2, b2)


def _keys_call(x, Wk, bk):
    grid = (N // _RB,)
    return pl.pallas_call(
        _keys_body,
        grid=grid,
        in_specs=[
            pl.BlockSpec((_RB, D_IN), lambda i: (i, 0)),
            pl.BlockSpec((D_IN, D_LAT), lambda i: (0, 0)),
            pl.BlockSpec((1, D_LAT), lambda i: (0, 0)),
        ],
        out_specs=pl.BlockSpec((_RB, D_LAT), lambda i: (i, 0)),
        out_shape=jax.ShapeDtypeStruct((N, D_LAT), jnp.float32),
    )(x, Wk, bk)


def kernel(x, edge, W_gnn, b_gnn, W1, b1, W2, b2, Wk, bk):
    src = edge[0].astype(jnp.int32)
    dst = edge[1].astype(jnp.int32)
    pad = EPAD - E
    srcp = jnp.concatenate([src, jnp.zeros((pad,), jnp.int32)])
    dstp = jnp.concatenate([dst, jnp.full((pad,), -1, jnp.int32)])
    x2 = x.reshape(N, NC, DH).reshape(N * NC, DH)
    src2 = (srcp[None, :] * NC
            + jnp.arange(NC, dtype=jnp.int32)[:, None]).reshape(NC * NS, CH, CK)
    dstf = jnp.where(dstp < 0, N, dstp)
    dst3 = dstf.reshape(NS, CH, CK)
    dstd = dstf.reshape(NS * NC, CHD, CK)

    ones_blk = jnp.ones((CK, DH), jnp.float32)
    zrow_blk = jnp.zeros((ZPT, DH), jnp.float32)
    agg = _sc_aggregate(x2, src2, dst3, zrow_blk)
    deg = _sc_degree(dstd, ones_blk, zrow_blk)

    keys = _keys_call(x, Wk, bk.reshape(1, D_LAT))
    z = _mlp_call(agg[0, :N], agg[1, :N], x,
                  deg[0, :N, 0:1], deg[1, :N, 0:1],
                  W_gnn, b_gnn.reshape(1, D_HID),
                  W1, b1.reshape(1, D_HID),
                  W2, b2.reshape(1, D_LAT))
    return (z, keys)


# speedup vs baseline: 2.6481x; 1.0228x over previous
import jax
import jax.numpy as jnp
from jax import lax
from jax.experimental import pallas as pl
from jax.experimental.pallas import tpu as pltpu
from jax.experimental.pallas import tpu_sc as plsc

N = 10000
E = 160000
D_IN = 256
D_HID = 512
D_LAT = 256

NC = 2
NS = 16
DH = D_IN // NC

CK = 128
CH = 80
EPAD = NS * CH * CK

ROWS_A = 10368
ZPT = ROWS_A // NS
NOUT = 10240
OPT = NOUT // NS

CHD = CH // NC
ROWS_D = 10368
ZPT_D = ROWS_D // NS
OPT_D = NOUT // NS


def _chunks(total, step):
    out = []
    off = 0
    while off < total:
        sz = min(step, total - off)
        out.append((off, sz))
        off += sz
    return tuple(out)


OCH = _chunks(OPT, CK)
OCHD = _chunks(OPT_D, CK)


def _agg_body(x2_hbm, src_hbm, dst_hbm, zrow_hbm, agg_hbm,
              src_v, dst_v, rows_v, agg_s):
    c = lax.axis_index("c")
    s = lax.axis_index("s")

    pltpu.sync_copy(src_hbm.at[c * NS + s], src_v)
    pltpu.sync_copy(dst_hbm.at[s], dst_v)
    pltpu.sync_copy(zrow_hbm, agg_s.at[pl.ds(s * ZPT, ZPT)])
    plsc.subcore_barrier()

    def body(ch, carry):
        pltpu.sync_copy(x2_hbm.at[src_v.at[ch]], rows_v)
        pltpu.sync_copy(rows_v, agg_s.at[dst_v.at[ch]], add=True)
        return carry

    lax.fori_loop(0, CH, body, None)
    plsc.subcore_barrier()

    for off, sz in OCH:
        lr = s * OPT + off
        pltpu.sync_copy(agg_s.at[pl.ds(lr, sz)],
                        agg_hbm.at[c, pl.ds(lr, sz)])


def _deg_body(dst_hbm, ones_hbm, zrow_hbm, deg_hbm,
              dst_v, ones_v, deg_s):
    c = lax.axis_index("c")
    s = lax.axis_index("s")

    pltpu.sync_copy(dst_hbm.at[s * NC + c], dst_v)
    pltpu.sync_copy(ones_hbm, ones_v)
    pltpu.sync_copy(zrow_hbm, deg_s.at[pl.ds(s * ZPT_D, ZPT_D)])
    plsc.subcore_barrier()

    def body(ch, carry):
        pltpu.sync_copy(ones_v, deg_s.at[dst_v.at[ch]], add=True)
        return carry

    lax.fori_loop(0, CHD, body, None)
    plsc.subcore_barrier()

    for off, sz in OCHD:
        lr = s * OPT_D + off
        pltpu.sync_copy(deg_s.at[pl.ds(lr, sz)],
                        deg_hbm.at[c, pl.ds(lr, sz)])


def _sc_aggregate(x2, src_i, dst_i, zrow_blk):
    mesh = plsc.VectorSubcoreMesh(core_axis_name="c", subcore_axis_name="s")
    f32 = jnp.float32
    run = pl.kernel(
        _agg_body,
        out_type=jax.ShapeDtypeStruct((NC, NOUT, DH), f32),
        mesh=mesh,
        scratch_types=[
            pltpu.VMEM((CH, CK), jnp.int32),
            pltpu.VMEM((CH, CK), jnp.int32),
            pltpu.VMEM((CK, DH), f32),
            pltpu.VMEM_SHARED((ROWS_A, DH), f32),
        ],
    )
    return run(x2, src_i, dst_i, zrow_blk)


def _sc_degree(dst_i, ones_blk, zrow_blk):
    mesh = plsc.VectorSubcoreMesh(core_axis_name="c", subcore_axis_name="s")
    f32 = jnp.float32
    run = pl.kernel(
        _deg_body,
        out_type=jax.ShapeDtypeStruct((NC, NS * OPT_D, DH), f32),
        mesh=mesh,
        scratch_types=[
            pltpu.VMEM((CHD, CK), jnp.int32),
            pltpu.VMEM((CK, DH), f32),
            pltpu.VMEM_SHARED((ROWS_D, DH), f32),
        ],
    )
    return run(dst_i, ones_blk, zrow_blk)


def _leaky(t):
    return jnp.where(t >= 0, t, 0.01 * t)


_PREC = lax.Precision.HIGHEST


def _mlp_body(a0_ref, a1_ref, x_ref, d0_ref, d1_ref,
              wg_ref, bg_ref, w1_ref, b1_ref, w2_ref, b2_ref,
              wk_ref, bk_ref, z_ref, keys_ref):
    x = x_ref[...]
    agg = jnp.concatenate([a0_ref[...], a1_ref[...]], axis=1)
    deg = d0_ref[...] + d1_ref[...]
    h = (agg + x) / (deg + 1.0)
    t = jnp.dot(h, wg_ref[...], precision=_PREC,
                preferred_element_type=jnp.float32) + bg_ref[...]
    t = _leaky(t)
    t = jnp.dot(t, w1_ref[...], precision=_PREC,
                preferred_element_type=jnp.float32) + b1_ref[...]
    t = _leaky(t)
    z_ref[...] = jnp.dot(t, w2_ref[...], precision=_PREC,
                         preferred_element_type=jnp.float32) + b2_ref[...]
    keys_ref[...] = jnp.dot(x, wk_ref[...], precision=_PREC,
                            preferred_element_type=jnp.float32) + bk_ref[...]


_RB = 1000


def _mlp_call(a0, a1, x, d0, d1, Wg, bg, W1, b1, W2, b2, Wk, bk):
    grid = (N // _RB,)
    row = lambda i: (i, 0)
    rep = lambda i: (0, 0)
    return pl.pallas_call(
        _mlp_body,
        grid=grid,
        in_specs=[
            pl.BlockSpec((_RB, DH), row),
            pl.BlockSpec((_RB, DH), row),
            pl.BlockSpec((_RB, D_IN), row),
            pl.BlockSpec((_RB, 1), row),
            pl.BlockSpec((_RB, 1), row),
            pl.BlockSpec((D_IN, D_HID), rep),
            pl.BlockSpec((1, D_HID), rep),
            pl.BlockSpec((D_HID, D_HID), rep),
            pl.BlockSpec((1, D_HID), rep),
            pl.BlockSpec((D_HID, D_LAT), rep),
            pl.BlockSpec((1, D_LAT), rep),
            pl.BlockSpec((D_IN, D_LAT), rep),
            pl.BlockSpec((1, D_LAT), rep),
        ],
        out_specs=[pl.BlockSpec((_RB, D_LAT), row),
                   pl.BlockSpec((_RB, D_LAT), row)],
        out_shape=[jax.ShapeDtypeStruct((N, D_LAT), jnp.float32),
                   jax.ShapeDtypeStruct((N, D_LAT), jnp.float32)],
    )(a0, a1, x, d0, d1, Wg, bg, W1, b1, W2, b2, Wk, bk)


def kernel(x, edge, W_gnn, b_gnn, W1, b1, W2, b2, Wk, bk):
    src = edge[0].astype(jnp.int32)
    dst = edge[1].astype(jnp.int32)
    pad = EPAD - E
    srcp = jnp.concatenate([src, jnp.zeros((pad,), jnp.int32)])
    dstp = jnp.concatenate([dst, jnp.full((pad,), -1, jnp.int32)])
    x2 = x.reshape(N, NC, DH).reshape(N * NC, DH)
    src2 = (srcp[None, :] * NC
            + jnp.arange(NC, dtype=jnp.int32)[:, None]).reshape(NC * NS, CH, CK)
    dstf = jnp.where(dstp < 0, N, dstp)
    dst3 = dstf.reshape(NS, CH, CK)
    dstd = dstf.reshape(NS * NC, CHD, CK)

    ones_blk = jnp.ones((CK, DH), jnp.float32)
    zrow_blk = jnp.zeros((ZPT, DH), jnp.float32)
    agg = _sc_aggregate(x2, src2, dst3, zrow_blk)
    deg = _sc_degree(dstd, ones_blk, zrow_blk)

    z, keys = _mlp_call(agg[0, :N], agg[1, :N], x,
                        deg[0, :N, 0:1], deg[1, :N, 0:1],
                        W_gnn, b_gnn.reshape(1, D_HID),
                        W1, b1.reshape(1, D_HID),
                        W2, b2.reshape(1, D_LAT),
                        Wk, bk.reshape(1, D_LAT))
    return (z, keys)
